# baseline (speedup 1.0000x reference)
import functools

import jax
import jax.numpy as jnp
from jax import lax
from jax.experimental import pallas as pl
from jax.experimental.pallas import tpu as pltpu
from jax.experimental.pallas import tpu_sc as plsc

B = 16
N = 20000
D_CLS = 80
D_BOX = 4
K = 1000
L = 16
NV = N // L
CHUNK = 128
INT_MIN = -2147483648


def _max_body(x_ref, o_ref):
    m = jnp.max(x_ref[0], axis=0)
    bits = lax.bitcast_convert_type(m, jnp.int32)
    xor_mask = lax.shift_right_arithmetic(bits, 31) | jnp.int32(INT_MIN)
    o_ref[0, 0, :] = (bits ^ xor_mask) ^ jnp.int32(-1)


def _conf_tc(pred_scores):
    scores_t = jnp.swapaxes(pred_scores, 1, 2)
    out3 = pl.pallas_call(
        _max_body,
        grid=(B,),
        in_specs=[pl.BlockSpec((1, D_CLS, N), lambda b: (b, 0, 0))],
        out_specs=pl.BlockSpec((1, 1, N), lambda b: (b, 0, 0)),
        out_shape=jax.ShapeDtypeStruct((B, 1, N), jnp.int32),
    )(scores_t)
    return out3.reshape(B, N)


def _sc_body(keys_hbm, scores_hbm, boxes_hbm, out_boxes, out_scores,
             keys_v, idxa, idxb, hist, fidx, fidx2, srows, brows, bout,
             sem1, sem2):
    c = lax.axis_index("c")
    s = lax.axis_index("s")
    b = s

    pltpu.sync_copy(keys_hbm.at[b], keys_v.at[pl.ds(0, N)])

    lanes = lax.iota(jnp.int32, L)
    ones = jnp.ones((L,), jnp.int32)
    keys_v[pl.ds(N, L)] = jnp.full((L,), -1, jnp.int32)

    def zero_hist():
        def zero_body(i, _):
            hist[pl.ds(i * L, L)] = jnp.zeros((L,), jnp.int32)
            return 0
        lax.fori_loop(0, 256, zero_body, 0)

    zero_hist()

    def hista_body(j, _):
        v = keys_v[pl.ds(j * L, L)]
        d = lax.shift_right_logical(v, 24)
        plsc.addupdate_scatter(hist, [d * L + lanes], ones)
        return 0

    lax.fori_loop(0, NV, hista_body, 0)

    def tsearch_body(i, carry):
        tot, tbin = carry
        snew = tot + jnp.sum(hist[pl.ds(i * L, L)])
        tbin = jnp.where((tot < K) & (snew >= K), i, tbin)
        return snew, tbin

    _, tbin = lax.fori_loop(0, 256, tsearch_body,
                            (jnp.int32(0), jnp.int32(255)))

    def compact_body(j, off):
        v = keys_v[pl.ds(j * L, L)]
        m = lax.shift_right_logical(v, 24) <= tbin
        plsc.store_compressed(idxa.at[pl.ds(off, L)], lanes + j * L, mask=m)
        return off + jnp.sum(m.astype(jnp.int32))

    cnum = lax.fori_loop(0, NV, compact_body, jnp.int32(0))
    idxa[pl.ds(cnum, L)] = jnp.full((L,), N, jnp.int32)
    ncv = (cnum + L - 1) // L

    for p in range(4):
        shift = 8 * p
        idx_in = (idxa, idxb, idxa, idxb)[p]
        idx_out = (idxb, idxa, idxb, idxa)[p]

        zero_hist()

        def hist_body(j, _, idx_in=idx_in, shift=shift):
            slots = lanes * ncv + j
            idx = plsc.load_gather(idx_in, [slots])
            k = plsc.load_gather(keys_v, [idx])
            d = lax.shift_right_logical(k, shift) & 0xFF
            plsc.addupdate_scatter(hist, [d * L + lanes], ones)
            return 0

        lax.fori_loop(0, ncv, hist_body, 0)

        def scan_body(i, carry):
            v = hist[pl.ds(i * L, L)]
            cum = plsc.cumsum(v)
            hist[pl.ds(i * L, L)] = cum - v + carry
            return carry + jnp.sum(v)

        lax.fori_loop(0, 256, scan_body, jnp.int32(0))

        def perm_body(j, _, idx_in=idx_in, idx_out=idx_out, shift=shift):
            slots = lanes * ncv + j
            idx = plsc.load_gather(idx_in, [slots])
            k = plsc.load_gather(keys_v, [idx])
            hslot = (lax.shift_right_logical(k, shift) & 0xFF) * L + lanes
            pos = plsc.load_gather(hist, [hslot])
            plsc.store_scatter(idx_out, [pos], idx)
            plsc.addupdate_scatter(hist, [hslot], ones)
            return 0

        lax.fori_loop(0, ncv, perm_body, 0)

    def process_chunk(r0, nval):
        def fill_body(t, _):
            fr = idxa[pl.ds(r0 + t * L, L)] + b * K
            fidx[pl.ds(t * L, L)] = fr
            fidx2[pl.ds(t * L, L)] = fr // 20
            return 0

        lax.fori_loop(0, CHUNK // L, fill_body, 0)
        cp1 = pltpu.async_copy(scores_hbm.at[fidx], srows, sem1)
        cp2 = pltpu.async_copy(boxes_hbm.at[fidx2], brows, sem2)
        cp1.wait()
        cp2.wait()

        def extract_body(t, _):
            colb = (fidx[pl.ds(t * L, L)] % 20) * 4
            rows = lanes + t * L
            for j in range(D_BOX):
                vals = plsc.load_gather(brows, [rows, colb + j])
                plsc.store_scatter(bout, [rows, lanes * 0 + j], vals)
            return 0

        lax.fori_loop(0, CHUNK // L, extract_body, 0)
        pltpu.sync_copy(srows.at[pl.ds(0, nval)],
                        out_scores.at[b, pl.ds(r0, nval)])
        pltpu.sync_copy(bout.at[pl.ds(0, nval)],
                        out_boxes.at[b, pl.ds(r0, nval)])

    for ch in range(3):
        process_chunk(c * 512 + ch * CHUNK, CHUNK)

    @pl.when(c == 0)
    def _():
        process_chunk(3 * CHUNK, CHUNK)

    @pl.when(c == 1)
    def _():
        process_chunk(512 + 3 * CHUNK, K - 512 - 3 * CHUNK)


@jax.jit
def kernel(pred_bboxes, pred_scores):
    keys = _conf_tc(pred_scores)
    scores_flat = pred_scores[:2].reshape(2 * N, D_CLS)
    boxes_flat = pred_bboxes[:2].reshape(2 * N * D_BOX // D_CLS, D_CLS)

    mesh = plsc.VectorSubcoreMesh(core_axis_name="c", subcore_axis_name="s")
    sc = pl.kernel(
        _sc_body,
        out_type=(
            jax.ShapeDtypeStruct((B, K, D_BOX), jnp.float32),
            jax.ShapeDtypeStruct((B, K, D_CLS), jnp.float32),
        ),
        mesh=mesh,
        compiler_params=pltpu.CompilerParams(
            needs_layout_passes=False, use_tc_tiling_on_sc=False),
        scratch_types=[
            pltpu.VMEM((N + L,), jnp.int32),
            pltpu.VMEM((N + L,), jnp.int32),
            pltpu.VMEM((N + L,), jnp.int32),
            pltpu.VMEM((256 * L,), jnp.int32),
            pltpu.VMEM((CHUNK,), jnp.int32),
            pltpu.VMEM((CHUNK,), jnp.int32),
            pltpu.VMEM((CHUNK, D_CLS), jnp.float32),
            pltpu.VMEM((CHUNK, D_CLS), jnp.float32),
            pltpu.VMEM((CHUNK, D_BOX), jnp.float32),
            pltpu.SemaphoreType.DMA,
            pltpu.SemaphoreType.DMA,
        ],
    )
    out_boxes, out_scores = sc(keys, scores_flat, boxes_flat)
    return out_boxes, out_scores


# speedup vs baseline: 1.0276x; 1.0276x over previous
import functools

import jax
import jax.numpy as jnp
from jax import lax
from jax.experimental import pallas as pl
from jax.experimental.pallas import tpu as pltpu
from jax.experimental.pallas import tpu_sc as plsc

B = 16
N = 20000
D_CLS = 80
D_BOX = 4
K = 1000
L = 16
NV = N // L
CHUNK = 128
INT_MIN = -2147483648


def _max_body(x_ref, o_ref):
    m = jnp.max(x_ref[0], axis=0)
    bits = lax.bitcast_convert_type(m, jnp.int32)
    xor_mask = lax.shift_right_arithmetic(bits, 31) | jnp.int32(INT_MIN)
    o_ref[0, 0, :] = (bits ^ xor_mask) ^ jnp.int32(-1)


def _conf_tc(pred_scores):
    scores_t = jnp.swapaxes(pred_scores, 1, 2)
    out3 = pl.pallas_call(
        _max_body,
        grid=(B,),
        in_specs=[pl.BlockSpec((1, D_CLS, N), lambda b: (b, 0, 0))],
        out_specs=pl.BlockSpec((1, 1, N), lambda b: (b, 0, 0)),
        out_shape=jax.ShapeDtypeStruct((B, 1, N), jnp.int32),
    )(scores_t)
    return out3.reshape(B, N)


def _sc_body(keys_hbm, scores_hbm, boxes_hbm, out_boxes, out_scores,
             keys_v, idxa, idxb, hist, fidx, fidx2, srows, brows, bout,
             sem1, sem2):
    c = lax.axis_index("c")
    s = lax.axis_index("s")
    b = s

    pltpu.sync_copy(keys_hbm.at[b], keys_v.at[pl.ds(0, N)])

    lanes = lax.iota(jnp.int32, L)
    ones = jnp.ones((L,), jnp.int32)
    keys_v[pl.ds(N, L)] = jnp.full((L,), -1, jnp.int32)

    def zero_hist():
        def zero_body(i, _):
            hist[pl.ds(i * L, L)] = jnp.zeros((L,), jnp.int32)
            return 0
        lax.fori_loop(0, 256, zero_body, 0, unroll=8)

    zero_hist()

    def hista_body(j, _):
        v = keys_v[pl.ds(j * L, L)]
        d = lax.shift_right_logical(v, 24)
        plsc.addupdate_scatter(hist, [d * L + lanes], ones)
        return 0

    lax.fori_loop(0, NV, hista_body, 0, unroll=10)

    def tsearch_body(i, carry):
        tot, tbin = carry
        snew = tot + jnp.sum(hist[pl.ds(i * L, L)])
        tbin = jnp.where((tot < K) & (snew >= K), i, tbin)
        return snew, tbin

    _, tbin = lax.fori_loop(0, 256, tsearch_body,
                            (jnp.int32(0), jnp.int32(255)), unroll=4)

    def compact_body(j, off):
        v = keys_v[pl.ds(j * L, L)]
        m = lax.shift_right_logical(v, 24) <= tbin
        plsc.store_compressed(idxa.at[pl.ds(off, L)], lanes + j * L, mask=m)
        return off + jnp.sum(m.astype(jnp.int32))

    cnum = lax.fori_loop(0, NV, compact_body, jnp.int32(0), unroll=5)
    idxa[pl.ds(cnum, L)] = jnp.full((L,), N, jnp.int32)
    ncv = (cnum + L - 1) // L

    for p in range(4):
        shift = 8 * p
        idx_in = (idxa, idxb, idxa, idxb)[p]
        idx_out = (idxb, idxa, idxb, idxa)[p]

        zero_hist()

        def hist_body(j, _, idx_in=idx_in, shift=shift):
            slots = lanes * ncv + j
            idx = plsc.load_gather(idx_in, [slots])
            k = plsc.load_gather(keys_v, [idx])
            d = lax.shift_right_logical(k, shift) & 0xFF
            plsc.addupdate_scatter(hist, [d * L + lanes], ones)
            return 0

        lax.fori_loop(0, ncv, hist_body, 0)

        def scan_body(i, carry):
            v = hist[pl.ds(i * L, L)]
            cum = plsc.cumsum(v)
            hist[pl.ds(i * L, L)] = cum - v + carry
            return carry + jnp.sum(v)

        lax.fori_loop(0, 256, scan_body, jnp.int32(0), unroll=4)

        def perm_body(j, _, idx_in=idx_in, idx_out=idx_out, shift=shift):
            slots = lanes * ncv + j
            idx = plsc.load_gather(idx_in, [slots])
            k = plsc.load_gather(keys_v, [idx])
            hslot = (lax.shift_right_logical(k, shift) & 0xFF) * L + lanes
            pos = plsc.load_gather(hist, [hslot])
            plsc.store_scatter(idx_out, [pos], idx)
            plsc.addupdate_scatter(hist, [hslot], ones)
            return 0

        lax.fori_loop(0, ncv, perm_body, 0)

    def process_chunk(r0, nval):
        def fill_body(t, _):
            fr = idxa[pl.ds(r0 + t * L, L)] + b * K
            fidx[pl.ds(t * L, L)] = fr
            fidx2[pl.ds(t * L, L)] = fr // 20
            return 0

        lax.fori_loop(0, CHUNK // L, fill_body, 0, unroll=4)
        cp1 = pltpu.async_copy(scores_hbm.at[fidx], srows, sem1)
        cp2 = pltpu.async_copy(boxes_hbm.at[fidx2], brows, sem2)
        cp1.wait()
        cp2.wait()

        def extract_body(t, _):
            colb = (fidx[pl.ds(t * L, L)] % 20) * 4
            rows = lanes + t * L
            for j in range(D_BOX):
                vals = plsc.load_gather(brows, [rows, colb + j])
                plsc.store_scatter(bout, [rows, lanes * 0 + j], vals)
            return 0

        lax.fori_loop(0, CHUNK // L, extract_body, 0, unroll=4)
        pltpu.sync_copy(srows.at[pl.ds(0, nval)],
                        out_scores.at[b, pl.ds(r0, nval)])
        pltpu.sync_copy(bout.at[pl.ds(0, nval)],
                        out_boxes.at[b, pl.ds(r0, nval)])

    for ch in range(3):
        process_chunk(c * 512 + ch * CHUNK, CHUNK)

    @pl.when(c == 0)
    def _():
        process_chunk(3 * CHUNK, CHUNK)

    @pl.when(c == 1)
    def _():
        process_chunk(512 + 3 * CHUNK, K - 512 - 3 * CHUNK)


@jax.jit
def kernel(pred_bboxes, pred_scores):
    keys = _conf_tc(pred_scores)
    scores_flat = pred_scores[:2].reshape(2 * N, D_CLS)
    boxes_flat = pred_bboxes[:2].reshape(2 * N * D_BOX // D_CLS, D_CLS)

    mesh = plsc.VectorSubcoreMesh(core_axis_name="c", subcore_axis_name="s")
    sc = pl.kernel(
        _sc_body,
        out_type=(
            jax.ShapeDtypeStruct((B, K, D_BOX), jnp.float32),
            jax.ShapeDtypeStruct((B, K, D_CLS), jnp.float32),
        ),
        mesh=mesh,
        compiler_params=pltpu.CompilerParams(
            needs_layout_passes=False, use_tc_tiling_on_sc=False),
        scratch_types=[
            pltpu.VMEM((N + L,), jnp.int32),
            pltpu.VMEM((N + L,), jnp.int32),
            pltpu.VMEM((N + L,), jnp.int32),
            pltpu.VMEM((256 * L,), jnp.int32),
            pltpu.VMEM((CHUNK,), jnp.int32),
            pltpu.VMEM((CHUNK,), jnp.int32),
            pltpu.VMEM((CHUNK, D_CLS), jnp.float32),
            pltpu.VMEM((CHUNK, D_CLS), jnp.float32),
            pltpu.VMEM((CHUNK, D_BOX), jnp.float32),
            pltpu.SemaphoreType.DMA,
            pltpu.SemaphoreType.DMA,
        ],
    )
    out_boxes, out_scores = sc(keys, scores_flat, boxes_flat)
    return out_boxes, out_scores


# speedup vs baseline: 1.0391x; 1.0112x over previous
import functools

import jax
import jax.numpy as jnp
from jax import lax
from jax.experimental import pallas as pl
from jax.experimental.pallas import tpu as pltpu
from jax.experimental.pallas import tpu_sc as plsc

B = 16
N = 20000
D_CLS = 80
D_BOX = 4
K = 1000
L = 16
NV = N // L
CHUNK = 128
INT_MIN = -2147483648


def _max_body(x_ref, o_ref):
    m = jnp.max(x_ref[0], axis=0)
    bits = lax.bitcast_convert_type(m, jnp.int32)
    xor_mask = lax.shift_right_arithmetic(bits, 31) | jnp.int32(INT_MIN)
    o_ref[0, 0, :] = (bits ^ xor_mask) ^ jnp.int32(-1)


def _conf_tc(pred_scores):
    scores_t = jnp.swapaxes(pred_scores, 1, 2)
    out3 = pl.pallas_call(
        _max_body,
        grid=(B,),
        in_specs=[pl.BlockSpec((1, D_CLS, N), lambda b: (b, 0, 0))],
        out_specs=pl.BlockSpec((1, 1, N), lambda b: (b, 0, 0)),
        out_shape=jax.ShapeDtypeStruct((B, 1, N), jnp.int32),
    )(scores_t)
    return out3.reshape(B, N)


def _sc_body(keys_hbm, scores_hbm, boxes_hbm, out_boxes, out_scores,
             keys_v, idxa, idxb, hist, fidx0, fidx1, fidx20, fidx21,
             srows0, srows1, brows0, brows1, bout,
             sem_s0, sem_s1, sem_b0, sem_b1):
    c = lax.axis_index("c")
    s = lax.axis_index("s")
    b = s

    pltpu.sync_copy(keys_hbm.at[b], keys_v.at[pl.ds(0, N)])

    lanes = lax.iota(jnp.int32, L)
    ones = jnp.ones((L,), jnp.int32)
    keys_v[pl.ds(N, L)] = jnp.full((L,), -1, jnp.int32)

    def zero_hist():
        def zero_body(i, _):
            hist[pl.ds(i * L, L)] = jnp.zeros((L,), jnp.int32)
            return 0
        lax.fori_loop(0, 256, zero_body, 0, unroll=8)

    scope = jax.named_scope
    with scope("phA_hist"):
        zero_hist()

    def hista_body(j, _):
        v = keys_v[pl.ds(j * L, L)]
        d = lax.shift_right_logical(v, 24)
        plsc.addupdate_scatter(hist, [d * L + lanes], ones)
        return 0

    with scope("phA_hist2"):
        lax.fori_loop(0, NV, hista_body, 0, unroll=10)

    def tsearch_body(i, carry):
        tot, tbin = carry
        snew = tot + jnp.sum(hist[pl.ds(i * L, L)])
        tbin = jnp.where((tot < K) & (snew >= K), i, tbin)
        return snew, tbin

    with scope("phA_tsearch"):
        _, tbin = lax.fori_loop(0, 256, tsearch_body,
                                (jnp.int32(0), jnp.int32(255)), unroll=4)

    def compact_body(j, off):
        v = keys_v[pl.ds(j * L, L)]
        m = lax.shift_right_logical(v, 24) <= tbin
        plsc.store_compressed(idxa.at[pl.ds(off, L)], lanes + j * L, mask=m)
        return off + jnp.sum(m.astype(jnp.int32))

    with scope("phB_compact"):
        padv = jnp.full((L,), N, jnp.int32)
        idxa[pl.ds(K, L)] = padv
        idxa[pl.ds(K + L, L)] = padv
        cnum = lax.fori_loop(0, NV, compact_body, jnp.int32(0), unroll=5)
        idxa[pl.ds(cnum, L)] = padv
        ncv = (cnum + L - 1) // L

    for p in range(4):
      with scope(f"phC_pass{p}"):
        shift = 8 * p
        idx_in = (idxa, idxb, idxa, idxb)[p]
        idx_out = (idxb, idxa, idxb, idxa)[p]

        zero_hist()

        def hist_body(j, _, idx_in=idx_in, shift=shift):
            slots = lanes * ncv + j
            idx = plsc.load_gather(idx_in, [slots])
            k = plsc.load_gather(keys_v, [idx])
            d = lax.shift_right_logical(k, shift) & 0xFF
            plsc.addupdate_scatter(hist, [d * L + lanes], ones)
            return 0

        lax.fori_loop(0, ncv, hist_body, 0)

        def scan_body(i, carry):
            v = hist[pl.ds(i * L, L)]
            cum = plsc.cumsum(v)
            hist[pl.ds(i * L, L)] = cum - v + carry
            return carry + jnp.sum(v)

        lax.fori_loop(0, 256, scan_body, jnp.int32(0), unroll=4)

        def perm_body(j, _, idx_in=idx_in, idx_out=idx_out, shift=shift):
            slots = lanes * ncv + j
            idx = plsc.load_gather(idx_in, [slots])
            k = plsc.load_gather(keys_v, [idx])
            hslot = (lax.shift_right_logical(k, shift) & 0xFF) * L + lanes
            pos = plsc.load_gather(hist, [hslot])
            plsc.store_scatter(idx_out, [pos], idx)
            plsc.addupdate_scatter(hist, [hslot], ones)
            return 0

        lax.fori_loop(0, ncv, perm_body, 0)

    fidxs, fidx2s = (fidx0, fidx1), (fidx20, fidx21)
    srowss, browss = (srows0, srows1), (brows0, brows1)
    sems_s, sems_b = (sem_s0, sem_s1), (sem_b0, sem_b1)

    def fill(r0, slot):
        fa, f2 = fidxs[slot], fidx2s[slot]

        def fill_body(t, _):
            fr = idxa[pl.ds(r0 + t * L, L)] + b * K
            fa[pl.ds(t * L, L)] = fr
            f2[pl.ds(t * L, L)] = fr // 20
            return 0

        lax.fori_loop(0, CHUNK // L, fill_body, 0, unroll=4)

    def start(slot):
        return (pltpu.async_copy(scores_hbm.at[fidxs[slot]], srowss[slot],
                                 sems_s[slot]),
                pltpu.async_copy(boxes_hbm.at[fidx2s[slot]], browss[slot],
                                 sems_b[slot]))

    def drain(slot, cps):
        cps[0].wait()
        cps[1].wait()
        fa, br = fidxs[slot], browss[slot]

        def extract_body(t, _):
            colb = (fa[pl.ds(t * L, L)] % 20) * 4
            rows = lanes + t * L
            for j in range(D_BOX):
                vals = plsc.load_gather(br, [rows, colb + j])
                plsc.store_scatter(bout, [rows, lanes * 0 + j], vals)
            return 0

        lax.fori_loop(0, CHUNK // L, extract_body, 0, unroll=4)

    def writeout(slot, r0, nval):
        pltpu.sync_copy(srowss[slot].at[pl.ds(0, nval)],
                        out_scores.at[b, pl.ds(r0, nval)])
        pltpu.sync_copy(bout.at[pl.ds(0, nval)],
                        out_boxes.at[b, pl.ds(r0, nval)])

    with scope("phD_gather"):
        r0s = [c * 512 + ch * CHUNK for ch in range(4)]
        fill(r0s[0], 0)
        cps = {0: start(0)}
        for ch in range(4):
            slot = ch % 2
            if ch + 1 < 4:
                fill(r0s[ch + 1], (ch + 1) % 2)
                cps[ch + 1] = start((ch + 1) % 2)
            drain(slot, cps[ch])
            if ch < 3:
                writeout(slot, r0s[ch], CHUNK)
            else:
                @pl.when(c == 0)
                def _(slot=slot):
                    writeout(slot, 3 * CHUNK, CHUNK)

                @pl.when(c == 1)
                def _(slot=slot):
                    writeout(slot, 512 + 3 * CHUNK, K - 512 - 3 * CHUNK)


@jax.jit
def kernel(pred_bboxes, pred_scores):
    keys = _conf_tc(pred_scores)
    scores_flat = pred_scores[:2].reshape(2 * N, D_CLS)
    boxes_flat = pred_bboxes[:2].reshape(2 * N * D_BOX // D_CLS, D_CLS)

    mesh = plsc.VectorSubcoreMesh(core_axis_name="c", subcore_axis_name="s")
    sc = pl.kernel(
        _sc_body,
        out_type=(
            jax.ShapeDtypeStruct((B, K, D_BOX), jnp.float32),
            jax.ShapeDtypeStruct((B, K, D_CLS), jnp.float32),
        ),
        mesh=mesh,
        compiler_params=pltpu.CompilerParams(
            needs_layout_passes=False, use_tc_tiling_on_sc=False),
        scratch_types=[
            pltpu.VMEM((N + L,), jnp.int32),
            pltpu.VMEM((N + L,), jnp.int32),
            pltpu.VMEM((N + L,), jnp.int32),
            pltpu.VMEM((256 * L,), jnp.int32),
            pltpu.VMEM((CHUNK,), jnp.int32),
            pltpu.VMEM((CHUNK,), jnp.int32),
            pltpu.VMEM((CHUNK,), jnp.int32),
            pltpu.VMEM((CHUNK,), jnp.int32),
            pltpu.VMEM((CHUNK, D_CLS), jnp.float32),
            pltpu.VMEM((CHUNK, D_CLS), jnp.float32),
            pltpu.VMEM((CHUNK, D_CLS), jnp.float32),
            pltpu.VMEM((CHUNK, D_CLS), jnp.float32),
            pltpu.VMEM((CHUNK, D_BOX), jnp.float32),
            pltpu.SemaphoreType.DMA,
            pltpu.SemaphoreType.DMA,
            pltpu.SemaphoreType.DMA,
            pltpu.SemaphoreType.DMA,
        ],
    )
    out_boxes, out_scores = sc(keys, scores_flat, boxes_flat)
    return out_boxes, out_scores


# speedup vs baseline: 1.0653x; 1.0252x over previous
import functools

import jax
import jax.numpy as jnp
from jax import lax
from jax.experimental import pallas as pl
from jax.experimental.pallas import tpu as pltpu
from jax.experimental.pallas import tpu_sc as plsc

B = 16
N = 20000
D_CLS = 80
D_BOX = 4
K = 1000
L = 16
NV = N // L
CHUNK = 128
INT_MIN = -2147483648


def _max_body(x_ref, o_ref, t_ref):
    m = jnp.max(x_ref[0], axis=0)
    bits = lax.bitcast_convert_type(m, jnp.int32)
    xor_mask = lax.shift_right_arithmetic(bits, 31) | jnp.int32(INT_MIN)
    o_ref[0, 0, :] = (bits ^ xor_mask) ^ jnp.int32(-1)

    @pl.when(pl.program_id(0) < 2)
    def _():
        t_ref[0] = x_ref[0].T


def _conf_tc(pred_scores):
    scores_t = jnp.swapaxes(pred_scores, 1, 2)
    out3, table = pl.pallas_call(
        _max_body,
        grid=(B,),
        in_specs=[pl.BlockSpec((1, D_CLS, N), lambda b: (b, 0, 0))],
        out_specs=[
            pl.BlockSpec((1, 1, N), lambda b: (b, 0, 0)),
            pl.BlockSpec((1, N, D_CLS), lambda b: (jnp.minimum(b, 1), 0, 0)),
        ],
        out_shape=[
            jax.ShapeDtypeStruct((B, 1, N), jnp.int32),
            jax.ShapeDtypeStruct((2, N, D_CLS), jnp.float32),
        ],
    )(scores_t)
    return out3.reshape(B, N), table.reshape(2 * N, D_CLS)


def _sc_body(keys_hbm, scores_hbm, boxes_hbm, out_boxes, out_scores,
             keys_v, idxa, idxb, hist, fidx0, fidx1, fidx20, fidx21,
             srows0, srows1, brows0, brows1, bout,
             sem_s0, sem_s1, sem_b0, sem_b1):
    c = lax.axis_index("c")
    s = lax.axis_index("s")
    b = s

    pltpu.sync_copy(keys_hbm.at[b], keys_v.at[pl.ds(0, N)])

    lanes = lax.iota(jnp.int32, L)
    ones = jnp.ones((L,), jnp.int32)
    keys_v[pl.ds(N, L)] = jnp.full((L,), -1, jnp.int32)

    def zero_hist():
        def zero_body(i, _):
            hist[pl.ds(i * L, L)] = jnp.zeros((L,), jnp.int32)
            return 0
        lax.fori_loop(0, 256, zero_body, 0, unroll=8)

    scope = jax.named_scope
    with scope("phA_hist"):
        zero_hist()

    def hista_body(j, _):
        v = keys_v[pl.ds(j * L, L)]
        d = lax.shift_right_logical(v, 24)
        plsc.addupdate_scatter(hist, [d * L + lanes], ones)
        return 0

    with scope("phA_hist2"):
        lax.fori_loop(0, NV, hista_body, 0, unroll=10)

    def tsearch_body(i, carry):
        tot, tbin = carry
        snew = tot + jnp.sum(hist[pl.ds(i * L, L)])
        tbin = jnp.where((tot < K) & (snew >= K), i, tbin)
        return snew, tbin

    with scope("phA_tsearch"):
        _, tbin = lax.fori_loop(0, 256, tsearch_body,
                                (jnp.int32(0), jnp.int32(255)), unroll=4)

    def compact_body(j, off):
        v = keys_v[pl.ds(j * L, L)]
        m = lax.shift_right_logical(v, 24) <= tbin
        plsc.store_compressed(idxa.at[pl.ds(off, L)], lanes + j * L, mask=m)
        return off + jnp.sum(m.astype(jnp.int32))

    with scope("phB_compact"):
        padv = jnp.full((L,), N, jnp.int32)
        idxa[pl.ds(K, L)] = padv
        idxa[pl.ds(K + L, L)] = padv
        cnum = lax.fori_loop(0, NV, compact_body, jnp.int32(0), unroll=5)
        idxa[pl.ds(cnum, L)] = padv
        ncv = (cnum + L - 1) // L

    for p in range(4):
      with scope(f"phC_pass{p}"):
        shift = 8 * p
        idx_in = (idxa, idxb, idxa, idxb)[p]
        idx_out = (idxb, idxa, idxb, idxa)[p]

        zero_hist()

        def hist_body(j, _, idx_in=idx_in, shift=shift):
            slots = lanes * ncv + j
            idx = plsc.load_gather(idx_in, [slots])
            k = plsc.load_gather(keys_v, [idx])
            d = lax.shift_right_logical(k, shift) & 0xFF
            plsc.addupdate_scatter(hist, [d * L + lanes], ones)
            return 0

        lax.fori_loop(0, ncv, hist_body, 0)

        def scan_body(i, carry):
            v = hist[pl.ds(i * L, L)]
            cum = plsc.cumsum(v)
            hist[pl.ds(i * L, L)] = cum - v + carry
            return carry + jnp.sum(v)

        lax.fori_loop(0, 256, scan_body, jnp.int32(0), unroll=4)

        def perm_body(j, _, idx_in=idx_in, idx_out=idx_out, shift=shift):
            slots = lanes * ncv + j
            idx = plsc.load_gather(idx_in, [slots])
            k = plsc.load_gather(keys_v, [idx])
            hslot = (lax.shift_right_logical(k, shift) & 0xFF) * L + lanes
            pos = plsc.load_gather(hist, [hslot])
            plsc.store_scatter(idx_out, [pos], idx)
            plsc.addupdate_scatter(hist, [hslot], ones)
            return 0

        lax.fori_loop(0, ncv, perm_body, 0)

    fidxs, fidx2s = (fidx0, fidx1), (fidx20, fidx21)
    srowss, browss = (srows0, srows1), (brows0, brows1)
    sems_s, sems_b = (sem_s0, sem_s1), (sem_b0, sem_b1)

    def fill(r0, slot):
        fa, f2 = fidxs[slot], fidx2s[slot]

        def fill_body(t, _):
            fr = idxa[pl.ds(r0 + t * L, L)] + b * K
            fa[pl.ds(t * L, L)] = fr
            f2[pl.ds(t * L, L)] = fr // 20
            return 0

        lax.fori_loop(0, CHUNK // L, fill_body, 0, unroll=4)

    def start(slot):
        return (pltpu.async_copy(scores_hbm.at[fidxs[slot]], srowss[slot],
                                 sems_s[slot]),
                pltpu.async_copy(boxes_hbm.at[fidx2s[slot]], browss[slot],
                                 sems_b[slot]))

    def drain(slot, cps):
        cps[0].wait()
        cps[1].wait()
        fa, br = fidxs[slot], browss[slot]

        def extract_body(t, _):
            colb = (fa[pl.ds(t * L, L)] % 20) * 4
            rows = lanes + t * L
            for j in range(D_BOX):
                vals = plsc.load_gather(br, [rows, colb + j])
                plsc.store_scatter(bout, [rows, lanes * 0 + j], vals)
            return 0

        lax.fori_loop(0, CHUNK // L, extract_body, 0, unroll=4)

    def writeout(slot, r0, nval):
        pltpu.sync_copy(srowss[slot].at[pl.ds(0, nval)],
                        out_scores.at[b, pl.ds(r0, nval)])
        pltpu.sync_copy(bout.at[pl.ds(0, nval)],
                        out_boxes.at[b, pl.ds(r0, nval)])

    with scope("phD_gather"):
        r0s = [c * 512 + ch * CHUNK for ch in range(4)]
        fill(r0s[0], 0)
        cps = {0: start(0)}
        for ch in range(4):
            slot = ch % 2
            if ch + 1 < 4:
                fill(r0s[ch + 1], (ch + 1) % 2)
                cps[ch + 1] = start((ch + 1) % 2)
            drain(slot, cps[ch])
            if ch < 3:
                writeout(slot, r0s[ch], CHUNK)
            else:
                @pl.when(c == 0)
                def _(slot=slot):
                    writeout(slot, 3 * CHUNK, CHUNK)

                @pl.when(c == 1)
                def _(slot=slot):
                    writeout(slot, 512 + 3 * CHUNK, K - 512 - 3 * CHUNK)


@jax.jit
def kernel(pred_bboxes, pred_scores):
    keys, scores_flat = _conf_tc(pred_scores)
    boxes_flat = pred_bboxes[:2].reshape(2 * N * D_BOX // D_CLS, D_CLS)

    mesh = plsc.VectorSubcoreMesh(core_axis_name="c", subcore_axis_name="s")
    sc = pl.kernel(
        _sc_body,
        out_type=(
            jax.ShapeDtypeStruct((B, K, D_BOX), jnp.float32),
            jax.ShapeDtypeStruct((B, K, D_CLS), jnp.float32),
        ),
        mesh=mesh,
        compiler_params=pltpu.CompilerParams(
            needs_layout_passes=False, use_tc_tiling_on_sc=False),
        scratch_types=[
            pltpu.VMEM((N + L,), jnp.int32),
            pltpu.VMEM((N + L,), jnp.int32),
            pltpu.VMEM((N + L,), jnp.int32),
            pltpu.VMEM((256 * L,), jnp.int32),
            pltpu.VMEM((CHUNK,), jnp.int32),
            pltpu.VMEM((CHUNK,), jnp.int32),
            pltpu.VMEM((CHUNK,), jnp.int32),
            pltpu.VMEM((CHUNK,), jnp.int32),
            pltpu.VMEM((CHUNK, D_CLS), jnp.float32),
            pltpu.VMEM((CHUNK, D_CLS), jnp.float32),
            pltpu.VMEM((CHUNK, D_CLS), jnp.float32),
            pltpu.VMEM((CHUNK, D_CLS), jnp.float32),
            pltpu.VMEM((CHUNK, D_BOX), jnp.float32),
            pltpu.SemaphoreType.DMA,
            pltpu.SemaphoreType.DMA,
            pltpu.SemaphoreType.DMA,
            pltpu.SemaphoreType.DMA,
        ],
    )
    out_boxes, out_scores = sc(keys, scores_flat, boxes_flat)
    return out_boxes, out_scores
